# baseline (speedup 1.0000x reference)
import functools

import jax
import jax.numpy as jnp
from jax.experimental import pallas as pl
from jax.experimental.pallas import tpu as pltpu

_COMMIT_W = 0.25


def _vq_body(x_ref, cb_ref, out_ref, ind_ref, loss_ref, perp_ref,
             cluster_acc, loss_acc, *, n_total, k, d, bn):
    i = pl.program_id(0)
    nsteps = pl.num_programs(0)

    x = x_ref[...]
    cb = cb_ref[...]

    x_sq = jnp.sum(x * x, axis=-1, keepdims=True)
    c_sq = jnp.sum(cb * cb, axis=-1)[None, :]
    scores = jax.lax.dot_general(x, cb, (((1,), (1,)), ((), ())),
                                 preferred_element_type=jnp.float32)
    d2 = x_sq + c_sq - 2.0 * scores
    dist = jnp.sqrt(jnp.clip(d2, 0.0, None))
    m = jnp.min(dist, axis=1, keepdims=True)
    mask = dist == m
    ohf = mask.astype(jnp.float32)
    total = jnp.sum(ohf)

    @pl.when(i == 0)
    def _init():
        cluster_acc[...] = jnp.zeros_like(cluster_acc)
        loss_acc[0, 0] = 0.0

    loss_acc[0, 0] += jnp.sum(m * m)

    @pl.when(total == jnp.float32(bn))
    def _fast():
        iota_col = jax.lax.broadcasted_iota(
            jnp.int32, (k, 1), 0).astype(jnp.float32)
        ind_f = jax.lax.dot_general(ohf, iota_col, (((1,), (0,)), ((), ())),
                                    precision=jax.lax.Precision.HIGHEST,
                                    preferred_element_type=jnp.float32)
        ind_ref[...] = ind_f.astype(jnp.int32).reshape(ind_ref.shape)
        out_ref[...] = jax.lax.dot_general(
            ohf, cb, (((1,), (0,)), ((), ())),
            preferred_element_type=jnp.float32)
        cluster_acc[...] += jnp.sum(ohf, axis=0, keepdims=True)

    @pl.when(total != jnp.float32(bn))
    def _slow():
        iota_k = jax.lax.broadcasted_iota(jnp.int32, dist.shape, 1)
        ind = jnp.min(jnp.where(mask, iota_k, jnp.int32(k)), axis=1)
        ohx = (iota_k == ind[:, None]).astype(jnp.float32)
        ind_ref[...] = ind.reshape(ind_ref.shape)
        out_ref[...] = jax.lax.dot_general(
            ohx, cb, (((1,), (0,)), ((), ())),
            preferred_element_type=jnp.float32)
        cluster_acc[...] += jnp.sum(ohx, axis=0, keepdims=True)

    @pl.when(i == nsteps - 1)
    def _fini():
        cs = cluster_acc[...]
        probs = cs / jnp.sum(cs)
        ent = -jnp.sum(probs * jnp.log(probs + 1e-10))
        perp_ref[0, 0] = jnp.exp(ent)
        loss_ref[0, 0] = loss_acc[0, 0] * ((1.0 + _COMMIT_W) / (n_total * d))


def kernel(z, codebook):
    b, t, d = z.shape
    k = codebook.shape[0]
    x = z.reshape(-1, d)
    n = x.shape[0]
    bn = 1024
    nb = n // bn

    body = functools.partial(_vq_body, n_total=n, k=k, d=d, bn=bn)
    out, ind3, loss, perp = pl.pallas_call(
        body,
        grid=(nb,),
        in_specs=[
            pl.BlockSpec((bn, d), lambda i: (i, 0)),
            pl.BlockSpec((k, d), lambda i: (0, 0)),
        ],
        out_specs=[
            pl.BlockSpec((bn, d), lambda i: (i, 0)),
            pl.BlockSpec((1, 1, bn), lambda i: (i, 0, 0)),
            pl.BlockSpec(memory_space=pltpu.SMEM),
            pl.BlockSpec(memory_space=pltpu.SMEM),
        ],
        out_shape=[
            jax.ShapeDtypeStruct((n, d), jnp.float32),
            jax.ShapeDtypeStruct((nb, 1, bn), jnp.int32),
            jax.ShapeDtypeStruct((1, 1), jnp.float32),
            jax.ShapeDtypeStruct((1, 1), jnp.float32),
        ],
        scratch_shapes=[
            pltpu.VMEM((1, k), jnp.float32),
            pltpu.SMEM((1, 1), jnp.float32),
        ],
        compiler_params=pltpu.CompilerParams(
            dimension_semantics=("arbitrary",)),
    )(x, codebook)

    return (out.reshape(b, t, d), ind3.reshape(b, t),
            loss[0, 0], perp[0, 0])


# speedup vs baseline: 1.6280x; 1.6280x over previous
import functools

import jax
import jax.numpy as jnp
from jax.experimental import pallas as pl
from jax.experimental.pallas import tpu as pltpu

_CODEBOOK_SIZE = 1024
_COMMIT_W = 0.25


def _vq_body(x_ref, cb_ref, out_ref, ind_ref, loss_ref, perp_ref,
             cluster_acc, loss_acc, csq_acc, *, n_total, k, d):
    i = pl.program_id(0)
    nsteps = pl.num_programs(0)

    x = x_ref[...]
    cb = cb_ref[...]

    @pl.when(i == 0)
    def _csq():
        csq_acc[...] = jnp.sum(cb * cb, axis=-1)[None, :]

    x_sq = jnp.sum(x * x, axis=-1, keepdims=True)
    c_sq = csq_acc[...]
    scores = jax.lax.dot_general(x, cb, (((1,), (1,)), ((), ())),
                                 preferred_element_type=jnp.float32)
    d2 = x_sq + c_sq - 2.0 * scores
    dist = jnp.sqrt(jnp.clip(d2, 0.0, None))
    m = jnp.min(dist, axis=1, keepdims=True)
    mask = dist == m
    iota_k = jax.lax.broadcasted_iota(jnp.int32, dist.shape, 1)
    ind = jnp.min(jnp.where(mask, iota_k, jnp.int32(k)), axis=1)

    oh = (iota_k == ind[:, None]).astype(jnp.float32)
    q = jax.lax.dot_general(oh, cb, (((1,), (0,)), ((), ())),
                            preferred_element_type=jnp.float32)
    out_ref[...] = q
    ind_ref[...] = ind.reshape(ind_ref.shape)

    @pl.when(i == 0)
    def _init():
        cluster_acc[...] = jnp.zeros_like(cluster_acc)
        loss_acc[0, 0] = 0.0

    diff = q - x
    loss_acc[0, 0] += jnp.sum(diff * diff)
    cluster_acc[...] += jnp.sum(oh, axis=0, keepdims=True)

    @pl.when(i == nsteps - 1)
    def _fini():
        cs = cluster_acc[...]
        probs = cs / jnp.sum(cs)
        ent = -jnp.sum(probs * jnp.log(probs + 1e-10))
        perp_ref[0, 0] = jnp.exp(ent)
        loss_ref[0, 0] = loss_acc[0, 0] * ((1.0 + _COMMIT_W) / (n_total * d))


def kernel(z, codebook):
    b, t, d = z.shape
    k = codebook.shape[0]
    x = z.reshape(-1, d)
    n = x.shape[0]
    bn = 2048
    nb = n // bn

    body = functools.partial(_vq_body, n_total=n, k=k, d=d)
    out, ind3, loss, perp = pl.pallas_call(
        body,
        grid=(nb,),
        in_specs=[
            pl.BlockSpec((bn, d), lambda i: (i, 0)),
            pl.BlockSpec((k, d), lambda i: (0, 0)),
        ],
        out_specs=[
            pl.BlockSpec((bn, d), lambda i: (i, 0)),
            pl.BlockSpec((1, 1, bn), lambda i: (i, 0, 0)),
            pl.BlockSpec(memory_space=pltpu.SMEM),
            pl.BlockSpec(memory_space=pltpu.SMEM),
        ],
        out_shape=[
            jax.ShapeDtypeStruct((n, d), jnp.float32),
            jax.ShapeDtypeStruct((nb, 1, bn), jnp.int32),
            jax.ShapeDtypeStruct((1, 1), jnp.float32),
            jax.ShapeDtypeStruct((1, 1), jnp.float32),
        ],
        scratch_shapes=[
            pltpu.VMEM((1, k), jnp.float32),
            pltpu.SMEM((1, 1), jnp.float32),
            pltpu.VMEM((1, k), jnp.float32),
        ],
        compiler_params=pltpu.CompilerParams(
            dimension_semantics=("arbitrary",)),
    )(x, codebook)

    return (out.reshape(b, t, d), ind3.reshape(b, t),
            loss[0, 0], perp[0, 0])


# speedup vs baseline: 1.6519x; 1.0147x over previous
import functools

import jax
import jax.numpy as jnp
from jax.experimental import pallas as pl
from jax.experimental.pallas import tpu as pltpu

_CODEBOOK_SIZE = 1024
_COMMIT_W = 0.25


def _vq_body(x_ref, cb_ref, out_ref, ind_ref, loss_ref, perp_ref,
             cluster_acc, loss_acc, csq_acc, *, n_total, k, d):
    i = pl.program_id(0)
    nsteps = pl.num_programs(0)

    x = x_ref[...]
    cb = cb_ref[...]

    @pl.when(i == 0)
    def _csq():
        csq_acc[...] = jnp.sum(cb * cb, axis=-1)[None, :]

    x_sq = jnp.sum(x * x, axis=-1, keepdims=True)
    c_sq = csq_acc[...]
    scores = jax.lax.dot_general(x, cb, (((1,), (1,)), ((), ())),
                                 preferred_element_type=jnp.float32)
    d2 = x_sq + c_sq - 2.0 * scores
    dist = jnp.sqrt(jnp.clip(d2, 0.0, None))
    m = jnp.min(dist, axis=1, keepdims=True)
    mask = dist == m
    iota_k = jax.lax.broadcasted_iota(jnp.int32, dist.shape, 1)
    ind = jnp.min(jnp.where(mask, iota_k, jnp.int32(k)), axis=1)

    oh = (iota_k == ind[:, None]).astype(jnp.float32)
    q = jax.lax.dot_general(oh, cb, (((1,), (0,)), ((), ())),
                            preferred_element_type=jnp.float32)
    out_ref[...] = q
    ind_ref[...] = ind.reshape(ind_ref.shape)

    @pl.when(i == 0)
    def _init():
        cluster_acc[...] = jnp.zeros_like(cluster_acc)
        loss_acc[0, 0] = 0.0

    diff = q - x
    loss_acc[0, 0] += jnp.sum(diff * diff)
    cluster_acc[...] += jnp.sum(oh, axis=0, keepdims=True)

    @pl.when(i == nsteps - 1)
    def _fini():
        cs = cluster_acc[...]
        probs = cs / jnp.sum(cs)
        ent = -jnp.sum(probs * jnp.log(probs + 1e-10))
        perp_ref[0, 0] = jnp.exp(ent)
        loss_ref[0, 0] = loss_acc[0, 0] * ((1.0 + _COMMIT_W) / (n_total * d))


def kernel(z, codebook):
    b, t, d = z.shape
    k = codebook.shape[0]
    x = z.reshape(-1, d)
    n = x.shape[0]
    bn = 4096
    nb = n // bn

    body = functools.partial(_vq_body, n_total=n, k=k, d=d)
    out, ind3, loss, perp = pl.pallas_call(
        body,
        grid=(nb,),
        in_specs=[
            pl.BlockSpec((bn, d), lambda i: (i, 0)),
            pl.BlockSpec((k, d), lambda i: (0, 0)),
        ],
        out_specs=[
            pl.BlockSpec((bn, d), lambda i: (i, 0)),
            pl.BlockSpec((1, 1, bn), lambda i: (i, 0, 0)),
            pl.BlockSpec(memory_space=pltpu.SMEM),
            pl.BlockSpec(memory_space=pltpu.SMEM),
        ],
        out_shape=[
            jax.ShapeDtypeStruct((n, d), jnp.float32),
            jax.ShapeDtypeStruct((nb, 1, bn), jnp.int32),
            jax.ShapeDtypeStruct((1, 1), jnp.float32),
            jax.ShapeDtypeStruct((1, 1), jnp.float32),
        ],
        scratch_shapes=[
            pltpu.VMEM((1, k), jnp.float32),
            pltpu.SMEM((1, 1), jnp.float32),
            pltpu.VMEM((1, k), jnp.float32),
        ],
        compiler_params=pltpu.CompilerParams(
            dimension_semantics=("arbitrary",)),
    )(x, codebook)

    return (out.reshape(b, t, d), ind3.reshape(b, t),
            loss[0, 0], perp[0, 0])
